# baseline (speedup 1.0000x reference)
import jax
import jax.numpy as jnp
from jax import lax
from jax.experimental import pallas as pl
from jax.experimental.pallas import tpu as pltpu
from jax.experimental.pallas import tpu_sc as plsc

_VOCAB = 1000000
_DIM = 64
_L = 16

_INFO = plsc.get_sparse_core_info()
_NC = _INFO.num_cores
_NS = _INFO.num_subcores
_NW = _NC * _NS

_ROWS = 16384
_B_TOTAL = _ROWS * 50
_B_PER_W = _B_TOTAL // _NW
_CHUNK = 400
_OROWS = _CHUNK // 50
_GRP = 80
_NG = _CHUNK // _GRP
_ITERS = _B_PER_W // _CHUNK


def _sc_body(emb_hbm, msk_hbm, idx_hbm, imp_hbm, mv_hbm, out_hbm,
             idx_v, g_v, emb_v, msk_v, imp_v, mv_v, sem_e, sem_m):
    wid = lax.axis_index("s") * _NC + lax.axis_index("c")
    base = wid * _B_PER_W
    obase = wid * (_B_PER_W // 50)

    pltpu.sync_copy(imp_hbm, imp_v)
    pltpu.sync_copy(mv_hbm, mv_v)

    def chunk_body(t, carry):
        row0 = base + t * _CHUNK
        pltpu.sync_copy(idx_hbm.at[pl.ds(row0, _CHUNK)], idx_v)

        for i in range(_CHUNK // _L):
            v = idx_v[pl.ds(i * _L, _L)]
            g = jnp.where(v == 0, _VOCAB - 1, v - 1)
            r, c = divmod(i * _L, _GRP)
            g_v[r, pl.ds(c, _L)] = g

        handles = []
        for r in range(_NG):
            handles.append(pltpu.async_copy(
                emb_hbm.at[g_v.at[r]], emb_v.at[pl.ds(r * _GRP, _GRP)],
                sem_e))
            handles.append(pltpu.async_copy(
                msk_hbm.at[g_v.at[r]], msk_v.at[pl.ds(r * _GRP, _GRP)],
                sem_m))
        for h in handles:
            h.wait()

        park = tuple(imp_v[pl.ds(16 * j, 16)] for j in range(4)) + \
            tuple(mv_v[pl.ds(16 * j, 16)] for j in range(4))

        def row_body(b, carry2):
            impc = carry2[0:4]
            mvc = carry2[4:8]
            mrow = msk_v[b, pl.ds(0, 16)]
            for j in range(4):
                e = emb_v[b, pl.ds(16 * j, 16)]
                bj = lax.shift_right_logical(mrow, 8 * j) & 3
                impf = (bj & 1).astype(jnp.float32)
                missf = lax.shift_right_logical(bj, 1).astype(jnp.float32)
                e = e + (impc[j] - e) * impf
                e = e + (mvc[j] - e) * missf
                emb_v[b, pl.ds(16 * j, 16)] = e
            return carry2

        lax.fori_loop(0, _CHUNK, row_body, park, unroll=False)

        for q in range(_OROWS):
            pltpu.sync_copy(emb_v.at[pl.ds(q * 50, 50)],
                            out_hbm.at[obase + t * _OROWS + q])
        return carry

    lax.fori_loop(0, _ITERS, chunk_body, 0, unroll=False)


@jax.jit
def _sc_gather(emb, msk_w, idx_flat, imp, mv):
    mesh = plsc.VectorSubcoreMesh(core_axis_name="c", subcore_axis_name="s")
    fn = pl.kernel(
        _sc_body,
        mesh=mesh,
        compiler_params=pltpu.CompilerParams(use_tc_tiling_on_sc=False),
        out_type=jax.ShapeDtypeStruct((_ROWS, 50, _DIM), jnp.float32),
        scratch_types=[
            pltpu.VMEM((_CHUNK,), jnp.int32),
            pltpu.VMEM((_NG, _GRP), jnp.int32),
            pltpu.VMEM((_CHUNK, _DIM), jnp.float32),
            pltpu.VMEM((_CHUNK, 16), jnp.int32),
            pltpu.VMEM((_DIM,), jnp.float32),
            pltpu.VMEM((_DIM,), jnp.float32),
            pltpu.SemaphoreType.DMA,
            pltpu.SemaphoreType.DMA,
        ],
    )
    return fn(emb, msk_w, idx_flat, imp, mv)


def kernel(idx, embedding, nan_mask, impute_values, missing_vector):
    idx_flat = idx.reshape(-1).astype(jnp.int32)
    m = nan_mask.reshape(_VOCAB, 4, 16)
    w = (m[:, 0].astype(jnp.int32)
         | (m[:, 1].astype(jnp.int32) << 8)
         | (m[:, 2].astype(jnp.int32) << 16)
         | (m[:, 3].astype(jnp.int32) << 24))
    row_ids = lax.broadcasted_iota(jnp.int32, (_VOCAB, 16), 0)
    mask_w = jnp.where(row_ids == _VOCAB - 1,
                       jnp.int32(0x02020202), w)
    imp = impute_values.astype(jnp.float32)
    mv = missing_vector.reshape(-1).astype(jnp.float32)
    return _sc_gather(embedding, mask_w, idx_flat, imp, mv)


# speedup vs baseline: 1.8075x; 1.8075x over previous
import jax
import jax.numpy as jnp
from jax import lax
from jax.experimental import pallas as pl
from jax.experimental.pallas import tpu as pltpu
from jax.experimental.pallas import tpu_sc as plsc

_VOCAB = 1000000
_DIM = 64
_L = 16

_INFO = plsc.get_sparse_core_info()
_NC = _INFO.num_cores
_NS = _INFO.num_subcores
_NW = _NC * _NS

_ROWS = 16384
_B_TOTAL = _ROWS * 50
_B_PER_W = _B_TOTAL // _NW
_CHUNK = 400
_OROWS = _CHUNK // 50
_GRP = 80
_NG = _CHUNK // _GRP
_ITERS = _B_PER_W // _CHUNK


def _sc_body(tab_hbm, idx_hbm, out_hbm,
             idx_v, g_v, emb0_v, emb1_v, sem_i, sem_g, sem_o):
    wid = lax.axis_index("s") * _NC + lax.axis_index("c")
    base = wid * _B_PER_W
    obase = wid * (_B_PER_W // 50)
    embs = (emb0_v, emb1_v)

    def load_idx(t):
        pltpu.sync_copy(idx_hbm.at[pl.ds(base + t * _CHUNK, _CHUNK)], idx_v)
        for i in range(_CHUNK // _L):
            v = idx_v[pl.ds(i * _L, _L)]
            g = jnp.where(v == 0, _VOCAB - 1, v - 1)
            r, c = divmod(i * _L, _GRP)
            g_v[r, pl.ds(c, _L)] = g

    def fire_gathers(buf):
        return [pltpu.async_copy(
            tab_hbm.at[g_v.at[r]], buf.at[pl.ds(r * _GRP, _GRP)], sem_g)
            for r in range(_NG)]

    def fire_out(t, buf):
        return [pltpu.async_copy(
            buf.at[pl.ds(q * 50, 50)], out_hbm.at[obase + t * _OROWS + q],
            sem_o) for q in range(_OROWS)]

    load_idx(0)
    for h in fire_gathers(embs[0]):
        h.wait()

    def chunk_body(t, carry):
        b_cur = lax.rem(t, 2)

        def do(parity):
            cur = embs[parity]
            nxt = embs[1 - parity]
            ohs = fire_out(t, cur)
            load_idx(t + 1)
            ghs = fire_gathers(nxt)
            for h in ohs:
                h.wait()
            for h in ghs:
                h.wait()

        @pl.when(b_cur == 0)
        def _():
            do(0)

        @pl.when(b_cur == 1)
        def _():
            do(1)

        return carry

    lax.fori_loop(0, _ITERS - 1, chunk_body, 0, unroll=False)

    last = (_ITERS - 1) % 2
    for h in fire_out(_ITERS - 1, embs[last]):
        h.wait()


@jax.jit
def _sc_gather(tab, idx_flat):
    mesh = plsc.VectorSubcoreMesh(core_axis_name="c", subcore_axis_name="s")
    fn = pl.kernel(
        _sc_body,
        mesh=mesh,
        compiler_params=pltpu.CompilerParams(use_tc_tiling_on_sc=False),
        out_type=jax.ShapeDtypeStruct((_ROWS, 50, _DIM), jnp.float32),
        scratch_types=[
            pltpu.VMEM((_CHUNK,), jnp.int32),
            pltpu.VMEM((_NG, _GRP), jnp.int32),
            pltpu.VMEM((_CHUNK, _DIM), jnp.float32),
            pltpu.VMEM((_CHUNK, _DIM), jnp.float32),
            pltpu.SemaphoreType.DMA,
            pltpu.SemaphoreType.DMA,
            pltpu.SemaphoreType.DMA,
        ],
    )
    return fn(tab, idx_flat)


def kernel(idx, embedding, nan_mask, impute_values, missing_vector):
    idx_flat = idx.reshape(-1).astype(jnp.int32)
    row_ids = lax.broadcasted_iota(jnp.int32, (_VOCAB, 1), 0)
    tab = jnp.where(nan_mask, impute_values[None, :], embedding)
    tab = jnp.where(row_ids == _VOCAB - 1,
                    missing_vector.reshape(1, _DIM), tab)
    return _sc_gather(tab, idx_flat)


# speedup vs baseline: 1.9285x; 1.0669x over previous
import jax
import jax.numpy as jnp
from jax import lax
from jax.experimental import pallas as pl
from jax.experimental.pallas import tpu as pltpu
from jax.experimental.pallas import tpu_sc as plsc

_VOCAB = 1000000
_DIM = 64
_L = 16

_INFO = plsc.get_sparse_core_info()
_NC = _INFO.num_cores
_NS = _INFO.num_subcores
_NW = _NC * _NS

_ROWS = 16384
_B_TOTAL = _ROWS * 50
_B_PER_W = _B_TOTAL // _NW
_CHUNK = 400
_OROWS = _CHUNK // 50
_GRP = 80
_NG = _CHUNK // _GRP
_ITERS = _B_PER_W // _CHUNK


def _sc_body(tab_hbm, idx_hbm, out_hbm,
             idx_v, g_v, emb0_v, emb1_v, sem_i, sem_g, sem_o):
    wid = lax.axis_index("s") * _NC + lax.axis_index("c")
    base = wid * _B_PER_W
    embs = (emb0_v, emb1_v)

    def load_idx(t):
        pltpu.sync_copy(idx_hbm.at[pl.ds(base + t * _CHUNK, _CHUNK)], idx_v)
        for i in range(_CHUNK // _L):
            v = idx_v[pl.ds(i * _L, _L)]
            g = jnp.where(v == 0, _VOCAB - 1, v - 1)
            r, c = divmod(i * _L, _GRP)
            g_v[r, pl.ds(c, _L)] = g

    def fire_gathers(buf):
        return [pltpu.async_copy(
            tab_hbm.at[g_v.at[r]], buf.at[pl.ds(r * _GRP, _GRP)], sem_g)
            for r in range(_NG)]

    def fire_out(t, buf):
        return [pltpu.async_copy(
            buf, out_hbm.at[pl.ds(base + t * _CHUNK, _CHUNK)], sem_o)]

    load_idx(0)
    for h in fire_gathers(embs[0]):
        h.wait()

    def chunk_body(t, carry):
        b_cur = lax.rem(t, 2)

        def do(parity):
            cur = embs[parity]
            nxt = embs[1 - parity]
            ohs = fire_out(t, cur)
            load_idx(t + 1)
            ghs = fire_gathers(nxt)
            for h in ohs:
                h.wait()
            for h in ghs:
                h.wait()

        @pl.when(b_cur == 0)
        def _():
            do(0)

        @pl.when(b_cur == 1)
        def _():
            do(1)

        return carry

    lax.fori_loop(0, _ITERS - 1, chunk_body, 0, unroll=False)

    last = (_ITERS - 1) % 2
    for h in fire_out(_ITERS - 1, embs[last]):
        h.wait()


@jax.jit
def _sc_gather(tab, idx_flat):
    mesh = plsc.VectorSubcoreMesh(core_axis_name="c", subcore_axis_name="s")
    fn = pl.kernel(
        _sc_body,
        mesh=mesh,
        compiler_params=pltpu.CompilerParams(use_tc_tiling_on_sc=False),
        out_type=jax.ShapeDtypeStruct((_B_TOTAL, _DIM), jnp.float32),
        scratch_types=[
            pltpu.VMEM((_CHUNK,), jnp.int32),
            pltpu.VMEM((_NG, _GRP), jnp.int32),
            pltpu.VMEM((_CHUNK, _DIM), jnp.float32),
            pltpu.VMEM((_CHUNK, _DIM), jnp.float32),
            pltpu.SemaphoreType.DMA,
            pltpu.SemaphoreType.DMA,
            pltpu.SemaphoreType.DMA,
        ],
    )
    return fn(tab, idx_flat)


def kernel(idx, embedding, nan_mask, impute_values, missing_vector):
    idx_flat = idx.reshape(-1).astype(jnp.int32)
    n = _VOCAB * _DIM
    emb_f = embedding.reshape(-1)
    msk_f = nan_mask.reshape(-1)
    imp_f = jnp.broadcast_to(impute_values.astype(jnp.float32)[None, :],
                             (_VOCAB, _DIM)).reshape(-1)
    mv_f = jnp.broadcast_to(missing_vector.reshape(1, _DIM),
                            (_VOCAB, _DIM)).reshape(-1)
    ids = lax.iota(jnp.int32, n)
    tab_f = jnp.where(msk_f, imp_f, emb_f)
    tab_f = jnp.where(ids >= _DIM * (_VOCAB - 1), mv_f, tab_f)
    out = _sc_gather(tab_f.reshape(_VOCAB, _DIM), idx_flat)
    return out.reshape(idx.shape + (_DIM,))
